# baseline (speedup 1.0000x reference)
import functools

import jax
import jax.numpy as jnp
from jax import lax
from jax.experimental import pallas as pl
from jax.experimental.pallas import tpu as pltpu
from jax.experimental.pallas import tpu_sc as plsc


def _gather_kernel(B, D, NW, b_per_w, C):
    mesh = plsc.VectorSubcoreMesh(core_axis_name="c", subcore_axis_name="s")
    n_chunks = b_per_w // C

    @functools.partial(
        pl.kernel,
        mesh=mesh,
        out_type=jax.ShapeDtypeStruct((B, D), jnp.float32),
        scratch_types=[
            pltpu.VMEM((b_per_w,), jnp.int32),
            pltpu.VMEM((2, C, D), jnp.float32),
            pltpu.SemaphoreType.DMA,
            pltpu.SemaphoreType.DMA,
            pltpu.SemaphoreType.DMA,
            pltpu.SemaphoreType.DMA,
        ],
    )
    def k(x_hbm, table_hbm, out_hbm, idx_v, rows_v, g0, g1, s0, s1):
        wid = lax.axis_index("s") * 2 + lax.axis_index("c")
        base = wid * b_per_w
        pltpu.sync_copy(x_hbm.at[pl.ds(base, b_per_w)], idx_v)

        gsems = (g0, g1)
        ssems = (s0, s1)

        def gather(c, b):
            return pltpu.make_async_copy(
                table_hbm.at[idx_v.at[pl.ds(c * C, C)]], rows_v.at[b], gsems[b]
            )

        def store(c, b):
            return pltpu.make_async_copy(
                rows_v.at[b], out_hbm.at[pl.ds(base + c * C, C)], ssems[b]
            )

        gather(0, 0).start()

        def body(c, _):
            for b in range(2):
                cc = c + b
                gather(cc, b).wait()
                store(cc, b).start()
                @pl.when(cc >= 1)
                def _():
                    store(cc - 1, 1 - b).wait()

                @pl.when(cc + 1 < n_chunks)
                def _():
                    gather(cc + 1, 1 - b).start()

            return _

        lax.fori_loop(0, n_chunks // 2, lambda c, u: body(c * 2, u), None)
        store(n_chunks - 1, (n_chunks - 1) % 2).wait()

    return k


def kernel(x, table):
    S, J = x.shape
    V, D = table.shape
    B = S * J
    NW = 32
    b_per_w = B // NW
    C = 64
    xf = x.reshape(B).astype(jnp.int32)
    out = _gather_kernel(B, D, NW, b_per_w, C)(xf, table)
    return out.reshape(S, J, D)


# speedup vs baseline: 3.3590x; 3.3590x over previous
import functools

import jax
import jax.numpy as jnp
from jax import lax
from jax.experimental import pallas as pl
from jax.experimental.pallas import tpu as pltpu
from jax.experimental.pallas import tpu_sc as plsc


def _gather_kernel(B, D, V, NW, b_per_w, C):
    mesh = plsc.VectorSubcoreMesh(core_axis_name="c", subcore_axis_name="s")
    n_chunks = b_per_w // C

    @functools.partial(
        pl.kernel,
        mesh=mesh,
        out_type=jax.ShapeDtypeStruct((B, D), jnp.float32),
        scratch_types=[
            pltpu.VMEM((b_per_w,), jnp.int32),
            pltpu.VMEM((2, C, D), jnp.float32),
            pltpu.SemaphoreType.DMA,
            pltpu.SemaphoreType.DMA,
            pltpu.SemaphoreType.DMA,
            pltpu.SemaphoreType.DMA,
        ],
    )
    def k(x_hbm, table_hbm, out_hbm, idx_v, rows_v, g0, g1, s0, s1):
        wid = lax.axis_index("s") * 2 + lax.axis_index("c")
        base = wid * b_per_w
        pltpu.sync_copy(x_hbm.at[pl.ds(base, b_per_w)], idx_v)

        gsems = (g0, g1)
        ssems = (s0, s1)

        def gather(c, b):
            return pltpu.make_async_copy(
                table_hbm.at[idx_v.at[pl.ds(c * C, C)]], rows_v.at[b], gsems[b]
            )

        def store(c, b):
            return pltpu.make_async_copy(
                rows_v.at[b], out_hbm.at[pl.ds(base + c * C, C)], ssems[b]
            )

        gather(0, 0).start()

        def body(c, _):
            for b in range(2):
                cc = c + b
                gather(cc, b).wait()
                store(cc, b).start()
                @pl.when(cc >= 1)
                def _():
                    store(cc - 1, 1 - b).wait()

                @pl.when(cc + 1 < n_chunks)
                def _():
                    gather(cc + 1, 1 - b).start()

            return _

        lax.fori_loop(0, n_chunks // 2, lambda c, u: body(c * 2, u), None)
        store(n_chunks - 1, (n_chunks - 1) % 2).wait()

    return k


def kernel(x, table):
    S, J = x.shape
    V, D = table.shape
    B = S * J
    NW = 32
    b_per_w = B // NW
    C = 64
    table_rep = jnp.tile(table, (NW, 1))
    xf = x.reshape(B).astype(jnp.int32)
    xf = xf + V * (jnp.arange(B, dtype=jnp.int32) // b_per_w)
    out = _gather_kernel(B, D, V, NW, b_per_w, C)(xf, table_rep)
    return out.reshape(S, J, D)
